# Initial kernel scaffold; baseline (speedup 1.0000x reference)
import jax
import jax.numpy as jnp
from jax.experimental import pallas as pl


def kernel(features, ys, xs, validation, backend_feature):
    raise NotImplementedError("write your pallas kernel here")


# speedup vs baseline: 2.2530x; 2.2530x over previous
import functools

import jax
import jax.numpy as jnp
from jax import lax
from jax.experimental import pallas as pl
from jax.experimental.pallas import tpu as pltpu
from jax.experimental.pallas import tpu_sc as plsc

B = 32
C = 512
P = 2048
MAX_H = 70
MAX_W = 40
HW = MAX_H * MAX_W
HWP = 2816
CHUNK = 80
NCHUNK = HW // CHUNK
CB = 256
NCB = HWP // CB
L = 16
I32MAX = 2147483647
I32MIN = -2147483648


def _transpose_body(x_ref, o_ref):
    o_ref[0] = x_ref[0].T


def _transpose_feats(features):
    cc = 256
    return pl.pallas_call(
        _transpose_body,
        grid=(B, C // cc),
        in_specs=[pl.BlockSpec((1, cc, P), lambda b, c: (b, c, 0))],
        out_specs=pl.BlockSpec((1, P, cc), lambda b, c: (b, 0, c)),
        out_shape=jax.ShapeDtypeStruct((B, P, C), jnp.float32),
    )(features)


def _sc_body(ys_hbm, xs_hbm, tab_hbm, pt_hbm, gath_hbm,
             ys_v, xs_v, cell_v, pt_v, ptc_v, buf_v, red_v, sem):
    b = lax.axis_index("c") * 16 + lax.axis_index("s")
    pltpu.sync_copy(ys_hbm.at[b], ys_v)
    pltpu.sync_copy(xs_hbm.at[b], xs_v)

    iota = lax.iota(jnp.int32, L)

    def mm_body(i, carry):
        mny, mxy, mnx, mxx = carry
        yv = ys_v[pl.ds(i * L, L)]
        xv = xs_v[pl.ds(i * L, L)]
        return (jnp.minimum(mny, yv), jnp.maximum(mxy, yv),
                jnp.minimum(mnx, xv), jnp.maximum(mxx, xv))

    big = jnp.full((L,), I32MAX, jnp.int32)
    small = jnp.full((L,), I32MIN, jnp.int32)
    mny, mxy, mnx, mxx = lax.fori_loop(
        0, P // L, mm_body, (big, small, big, small))

    def _allreduce(v, op):
        for s in (8, 4, 2, 1):
            red_v[...] = v
            g = plsc.load_gather(red_v, [jnp.bitwise_and(iota + s, L - 1)])
            v = op(v, g)
        return v

    min_y = _allreduce(mny, jnp.minimum)
    max_y = _allreduce(mxy, jnp.maximum)
    min_x = _allreduce(mnx, jnp.minimum)
    max_x = _allreduce(mxx, jnp.maximum)
    h = max_y - min_y + 1
    w = max_x - min_x + 1
    one = jnp.full((L,), 1, jnp.int32)
    zero = jnp.full((L,), 0, jnp.int32)
    si = jnp.where(w > h, one, zero)
    h2 = si * w + (one - si) * h
    w2 = si * h + (one - si) * w
    ofh = (MAX_H - h2 + 1) // 2
    ofw = (MAX_W - w2 + 1) // 2

    def cell_body(i, _):
        yv = ys_v[pl.ds(i * L, L)] - min_y
        xv = xs_v[pl.ds(i * L, L)] - min_x
        iout = si * xv + (1 - si) * yv + ofh
        jout = si * yv + (1 - si) * xv + ofw
        cell_v[pl.ds(i * L, L)] = iout * MAX_W + jout
        return 0

    lax.fori_loop(0, P // L, cell_body, 0)

    def init_body(i, _):
        pt_v[pl.ds(i * L, L)] = jnp.full((L,), jnp.int32(-1))
        return 0

    lax.fori_loop(0, HWP // L, init_body, 0)

    perms = [jnp.bitwise_and(iota + r, L - 1) for r in range(1, L)]
    vmasks = [iota < (L - r) for r in range(1, L)]

    def dedup_body(i, _):
        base = i * L
        c = cell_v[pl.ds(base, L)]
        dup = iota < 0
        for r in range(1, L):
            g = plsc.load_gather(cell_v, [base + perms[r - 1]])
            dup = jnp.logical_or(
                dup, jnp.logical_and(g == c, vmasks[r - 1]))
        plsc.store_scatter(pt_v, [c], base + iota,
                           mask=jnp.logical_not(dup))
        return 0

    lax.fori_loop(0, P // L, dedup_body, 0)

    pltpu.sync_copy(pt_v, pt_hbm.at[b])

    boff = b * P

    def clamp_body(i, _):
        v = pt_v[pl.ds(i * L, L)]
        ptc_v[pl.ds(i * L, L)] = jnp.maximum(v, 0) + boff
        return 0

    lax.fori_loop(0, HW // L, clamp_body, 0)

    def gath_body(g, _):
        idx = ptc_v.at[pl.ds(g * CHUNK, CHUNK)]
        pltpu.async_copy(tab_hbm.at[idx], buf_v, sem).wait()
        pltpu.sync_copy(buf_v, gath_hbm.at[b, pl.ds(g * CHUNK, CHUNK)])
        return 0

    lax.fori_loop(0, NCHUNK, gath_body, 0)


_sc_mesh = plsc.VectorSubcoreMesh(core_axis_name="c", subcore_axis_name="s")

_sc_call = functools.partial(
    pl.kernel,
    out_type=(
        jax.ShapeDtypeStruct((B, HWP), jnp.int32),
        jax.ShapeDtypeStruct((B, HWP, C), jnp.float32),
    ),
    mesh=_sc_mesh,
    compiler_params=pltpu.CompilerParams(needs_layout_passes=False),
    scratch_types=[
        pltpu.VMEM((P,), jnp.int32),
        pltpu.VMEM((P,), jnp.int32),
        pltpu.VMEM((P,), jnp.int32),
        pltpu.VMEM((HWP,), jnp.int32),
        pltpu.VMEM((HW,), jnp.int32),
        pltpu.VMEM((CHUNK, C), jnp.float32),
        pltpu.VMEM((L,), jnp.int32),
        pltpu.SemaphoreType.DMA,
    ],
)(_sc_body)


def _finish_body(g_ref, pt_ref, bk_ref, o_ref):
    x = g_ref[0]
    pt = pt_ref[0, 0, 0]
    valid = (pt >= 0) & jnp.all(x != -1.0, axis=1)
    o_ref[0] = jnp.where(valid[None, :], x.T, bk_ref[...])


def _finish(gath, pt, backend_feature):
    ptr = pt.reshape(B, NCB, 1, CB)
    bk2 = backend_feature.reshape(C, 1)
    out = pl.pallas_call(
        _finish_body,
        grid=(B, NCB),
        in_specs=[
            pl.BlockSpec((1, CB, C), lambda b, j: (b, j, 0)),
            pl.BlockSpec((1, 1, 1, CB), lambda b, j: (b, j, 0, 0)),
            pl.BlockSpec((C, 1), lambda b, j: (0, 0)),
        ],
        out_specs=pl.BlockSpec((1, C, CB), lambda b, j: (b, 0, j)),
        out_shape=jax.ShapeDtypeStruct((B, C, HWP), jnp.float32),
    )(gath, ptr, bk2)
    return out[:, :, :HW].reshape(B, C, MAX_H, MAX_W)


def kernel(features, ys, xs, validation, backend_feature):
    feats = features.astype(jnp.float32)
    ysi = ys.astype(jnp.int32)
    xsi = xs.astype(jnp.int32)
    featT = _transpose_feats(feats)
    tab = featT.reshape(B * P, C)
    pt, gath = _sc_call(ysi, xsi, tab)
    return _finish(gath, pt, backend_feature.astype(jnp.float32))


# speedup vs baseline: 2.6241x; 1.1647x over previous
import functools

import jax
import jax.numpy as jnp
from jax import lax
from jax.experimental import pallas as pl
from jax.experimental.pallas import tpu as pltpu
from jax.experimental.pallas import tpu_sc as plsc

B = 32
C = 512
P = 2048
MAX_H = 70
MAX_W = 40
HW = MAX_H * MAX_W
CHUNK = 56
NCHUNK = HW // CHUNK
CC = 256
FC = 128
L = 16
I32MAX = 2147483647
I32MIN = -2147483648


def _transpose_body(x_ref, o_ref, m_ref):
    c = pl.program_id(1)
    x = x_ref[0]
    o_ref[0] = x.T
    m = jnp.all(x != -1.0, axis=0).astype(jnp.int32)

    @pl.when(c == 0)
    def _():
        m_ref[0, 0] = m

    @pl.when(c != 0)
    def _():
        m_ref[0, 0] = m_ref[0, 0] & m


def _transpose_feats(features):
    return pl.pallas_call(
        _transpose_body,
        grid=(B, C // CC),
        in_specs=[pl.BlockSpec((1, CC, P), lambda b, c: (b, c, 0))],
        out_specs=[
            pl.BlockSpec((1, P, CC), lambda b, c: (b, 0, c)),
            pl.BlockSpec((1, 1, P), lambda b, c: (b, 0, 0)),
        ],
        out_shape=[
            jax.ShapeDtypeStruct((B, P, C), jnp.float32),
            jax.ShapeDtypeStruct((B, 1, P), jnp.int32),
        ],
    )(features)


def _sc_body(ys_hbm, xs_hbm, tab_hbm, rm_hbm, val_hbm, gath_hbm,
             ys_v, xs_v, cell_v, pt_v, ptc_v, rm_v, val_v,
             buf0, buf1, sem0, sem1):
    b = lax.axis_index("c") * 16 + lax.axis_index("s")
    pltpu.sync_copy(ys_hbm.at[b], ys_v)
    pltpu.sync_copy(xs_hbm.at[b], xs_v)
    pltpu.sync_copy(rm_hbm.at[b], rm_v)

    iota = lax.iota(jnp.int32, L)

    def mm_body(i, carry):
        mny, mxy, mnx, mxx = carry
        yv = ys_v[pl.ds(i * L, L)]
        xv = xs_v[pl.ds(i * L, L)]
        return (jnp.minimum(mny, yv), jnp.maximum(mxy, yv),
                jnp.minimum(mnx, xv), jnp.maximum(mxx, xv))

    big = jnp.full((L,), I32MAX, jnp.int32)
    small = jnp.full((L,), I32MIN, jnp.int32)
    mny, mxy, mnx, mxx = lax.fori_loop(
        0, P // L, mm_body, (big, small, big, small))

    def _allreduce(v, op):
        for s in (8, 4, 2, 1):
            ptc_v[pl.ds(0, L)] = v
            g = plsc.load_gather(ptc_v, [jnp.bitwise_and(iota + s, L - 1)])
            v = op(v, g)
        return v

    min_y = _allreduce(mny, jnp.minimum)
    max_y = _allreduce(mxy, jnp.maximum)
    min_x = _allreduce(mnx, jnp.minimum)
    max_x = _allreduce(mxx, jnp.maximum)
    h = max_y - min_y + 1
    w = max_x - min_x + 1
    one = jnp.full((L,), 1, jnp.int32)
    zero = jnp.full((L,), 0, jnp.int32)
    si = jnp.where(w > h, one, zero)
    h2 = si * w + (one - si) * h
    w2 = si * h + (one - si) * w
    ofh = (MAX_H - h2 + 1) // 2
    ofw = (MAX_W - w2 + 1) // 2

    def cell_body(i, _):
        yv = ys_v[pl.ds(i * L, L)] - min_y
        xv = xs_v[pl.ds(i * L, L)] - min_x
        iout = si * xv + (one - si) * yv + ofh
        jout = si * yv + (one - si) * xv + ofw
        cell_v[pl.ds(i * L, L)] = iout * MAX_W + jout
        return 0

    lax.fori_loop(0, P // L, cell_body, 0)

    def init_body(i, _):
        pt_v[pl.ds(i * L, L)] = jnp.full((L,), jnp.int32(-1))
        return 0

    lax.fori_loop(0, HW // L, init_body, 0)

    perms = [jnp.bitwise_and(iota + r, L - 1) for r in range(1, L)]
    vmasks = [iota < (L - r) for r in range(1, L)]

    def dedup_body(i, _):
        base = i * L
        c = cell_v[pl.ds(base, L)]
        dup = iota < 0
        for r in range(1, L):
            g = plsc.load_gather(cell_v, [base + perms[r - 1]])
            dup = jnp.logical_or(
                dup, jnp.logical_and(g == c, vmasks[r - 1]))
        plsc.store_scatter(pt_v, [c], base + iota,
                           mask=jnp.logical_not(dup))
        return 0

    lax.fori_loop(0, P // L, dedup_body, 0)

    boff = b * P

    def clamp_body(i, _):
        v = pt_v[pl.ds(i * L, L)]
        vc = jnp.maximum(v, 0)
        rm = plsc.load_gather(rm_v, [vc])
        ok = jnp.logical_and(v >= 0, rm != 0)
        val_v[pl.ds(i * L, L)] = jnp.where(ok, one, zero)
        ptc_v[pl.ds(i * L, L)] = vc + boff
        return 0

    lax.fori_loop(0, HW // L, clamp_body, 0)

    pltpu.sync_copy(val_v, val_hbm.at[b])

    def _start(g, buf, sem):
        idx = ptc_v.at[pl.ds(g * CHUNK, CHUNK)]
        pltpu.async_copy(tab_hbm.at[idx], buf, sem)

    def _drain(buf, sem):
        pltpu.make_async_copy(tab_hbm.at[pl.ds(0, CHUNK)], buf, sem).wait()

    _start(0, buf0, sem0)

    def gath_body(i, _):
        g0 = i * 2
        g1 = g0 + 1
        _start(g1, buf1, sem1)
        _drain(buf0, sem0)
        pltpu.sync_copy(buf0, gath_hbm.at[b, pl.ds(g0 * CHUNK, CHUNK)])

        @pl.when(g1 + 1 < NCHUNK)
        def _():
            _start(g1 + 1, buf0, sem0)

        _drain(buf1, sem1)
        pltpu.sync_copy(buf1, gath_hbm.at[b, pl.ds(g1 * CHUNK, CHUNK)])
        return 0

    lax.fori_loop(0, NCHUNK // 2, gath_body, 0)


_sc_mesh = plsc.VectorSubcoreMesh(core_axis_name="c", subcore_axis_name="s")

_sc_call = functools.partial(
    pl.kernel,
    out_type=(
        jax.ShapeDtypeStruct((B, HW), jnp.int32),
        jax.ShapeDtypeStruct((B, HW, C), jnp.float32),
    ),
    mesh=_sc_mesh,
    compiler_params=pltpu.CompilerParams(needs_layout_passes=False),
    scratch_types=[
        pltpu.VMEM((P,), jnp.int32),
        pltpu.VMEM((P,), jnp.int32),
        pltpu.VMEM((P,), jnp.int32),
        pltpu.VMEM((HW,), jnp.int32),
        pltpu.VMEM((HW,), jnp.int32),
        pltpu.VMEM((P,), jnp.int32),
        pltpu.VMEM((HW,), jnp.int32),
        pltpu.VMEM((CHUNK, C), jnp.float32),
        pltpu.VMEM((CHUNK, C), jnp.float32),
        pltpu.SemaphoreType.DMA,
        pltpu.SemaphoreType.DMA,
    ],
)(_sc_body)


def _finish_body(eye_ref, g_ref, v_ref, bk_ref, o_ref):
    x = g_ref[0]
    v = v_ref[0, 0] != 0
    xt = lax.dot_general(
        eye_ref[...], x, (((1,), (1,)), ((), ())),
        preferred_element_type=jnp.float32,
        precision=lax.Precision.HIGHEST)
    o_ref[0] = jnp.where(v[None, :], xt, bk_ref[...])


def _finish(gath, valid, backend_feature, eye):
    vr = valid.reshape(B, 1, HW)
    bk2 = backend_feature.reshape(C, 1)
    out = pl.pallas_call(
        _finish_body,
        grid=(B, C // FC),
        in_specs=[
            pl.BlockSpec((FC, FC), lambda b, c: (0, 0)),
            pl.BlockSpec((1, HW, FC), lambda b, c: (b, 0, c)),
            pl.BlockSpec((1, 1, HW), lambda b, c: (b, 0, 0)),
            pl.BlockSpec((FC, 1), lambda b, c: (c, 0)),
        ],
        out_specs=pl.BlockSpec((1, FC, HW), lambda b, c: (b, c, 0)),
        out_shape=jax.ShapeDtypeStruct((B, C, HW), jnp.float32),
    )(eye, gath, vr, bk2)
    return out.reshape(B, C, MAX_H, MAX_W)


def kernel(features, ys, xs, validation, backend_feature):
    feats = features.astype(jnp.float32)
    ysi = ys.astype(jnp.int32)
    xsi = xs.astype(jnp.int32)
    featT, rowmask = _transpose_feats(feats)
    tab = featT.reshape(B * P, C)
    valid, gath = _sc_call(ysi, xsi, tab, rowmask.reshape(B, P))
    eye = jnp.eye(FC, dtype=jnp.float32)
    return _finish(gath, valid, backend_feature.astype(jnp.float32), eye)
